# baseline (speedup 1.0000x reference)
import functools

import jax
import jax.numpy as jnp
from jax import lax
from jax.experimental import pallas as pl
from jax.experimental.pallas import tpu as pltpu
from jax.experimental.pallas import tpu_sc as plsc

NUM_USERS = 1000000
NUM_ITEMS = 1000000
EMB = 64
BATCH = 16384

NC = 2
NS = 16
NW = NC * NS
LANES = 16
B_PER_W = BATCH // NW
IDX_CHUNK = 128
N_IDX_CHUNKS = B_PER_W // IDX_CHUNK
ROW_PAIR = 2 * EMB
N_PASS = 2
B_PER_PASS = B_PER_W // N_PASS


def _sc_kernel_body(uidx_hbm, iidx_hbm, uemb2_hbm, iemb2_hbm, ub_hbm, ib_hbm,
                    out_hbm,
                    uidx_v, iidx_v, tu_v, ti_v, urows_v, irows_v,
                    ubv, ibv, out_v, sem):
    wid = lax.axis_index("s") * NC + lax.axis_index("c")
    base = pl.multiple_of(wid * B_PER_W, B_PER_W)

    pltpu.sync_copy(uidx_hbm.at[pl.ds(base, B_PER_W)], uidx_v)
    pltpu.sync_copy(iidx_hbm.at[pl.ds(base, B_PER_W)], iidx_v)

    bias_copies = []
    for k in range(N_IDX_CHUNKS):
        s = pl.ds(k * IDX_CHUNK, IDX_CHUNK)
        bias_copies.append(pltpu.async_copy(ub_hbm.at[uidx_v.at[s]],
                                            ubv.at[s], sem))
        bias_copies.append(pltpu.async_copy(ib_hbm.at[iidx_v.at[s]],
                                            ibv.at[s], sem))

    def halver(j, _):
        s = pl.ds(pl.multiple_of(j * LANES, LANES), LANES)
        tu_v[s] = lax.shift_right_logical(uidx_v[s], 1)
        ti_v[s] = lax.shift_right_logical(iidx_v[s], 1)
        return 0

    lax.fori_loop(0, B_PER_W // LANES, halver, 0)

    for c in bias_copies:
        c.wait()

    for p in range(N_PASS):
        poff = p * B_PER_PASS
        copies = []
        for k in range(B_PER_PASS // IDX_CHUNK):
            s_idx = pl.ds(poff + k * IDX_CHUNK, IDX_CHUNK)
            s_dst = pl.ds(k * IDX_CHUNK, IDX_CHUNK)
            copies.append(pltpu.async_copy(uemb2_hbm.at[tu_v.at[s_idx]],
                                           urows_v.at[s_dst], sem))
            copies.append(pltpu.async_copy(iemb2_hbm.at[ti_v.at[s_idx]],
                                           irows_v.at[s_dst], sem))
        for c in copies:
            c.wait()

        def chunk(j, _):
            start = pl.multiple_of(j * LANES, LANES)
            lanes = lax.iota(jnp.int32, LANES) + start
            gs = pl.ds(poff + start, LANES)
            uoff = (uidx_v[gs] & 1) * EMB
            ioff = (iidx_v[gs] & 1) * EMB
            acc = ubv[gs] + ibv[gs]
            for d in range(EMB):
                u = plsc.load_gather(urows_v, [lanes, uoff + d])
                v = plsc.load_gather(irows_v, [lanes, ioff + d])
                acc = acc + u * v
            out_v[gs] = acc
            return 0

        lax.fori_loop(0, B_PER_PASS // LANES, chunk, 0)

    pltpu.sync_copy(out_v, out_hbm.at[pl.ds(base, B_PER_W)])


@jax.jit
def _run(uidx, iidx, uemb, iemb, ub, ib):
    mesh = plsc.VectorSubcoreMesh(core_axis_name="c", subcore_axis_name="s")
    f = pl.kernel(
        _sc_kernel_body, mesh=mesh,
        out_type=jax.ShapeDtypeStruct((BATCH,), jnp.float32),
        scratch_types=[
            pltpu.VMEM((B_PER_W,), jnp.int32),
            pltpu.VMEM((B_PER_W,), jnp.int32),
            pltpu.VMEM((B_PER_W,), jnp.int32),
            pltpu.VMEM((B_PER_W,), jnp.int32),
            pltpu.VMEM((B_PER_PASS, ROW_PAIR), jnp.float32),
            pltpu.VMEM((B_PER_PASS, ROW_PAIR), jnp.float32),
            pltpu.VMEM((B_PER_W,), jnp.float32),
            pltpu.VMEM((B_PER_W,), jnp.float32),
            pltpu.VMEM((B_PER_W,), jnp.float32),
            pltpu.SemaphoreType.DMA,
        ],
        compiler_params=pltpu.CompilerParams(needs_layout_passes=False,
                                             use_tc_tiling_on_sc=False),
    )
    return f(uidx, iidx, uemb, iemb, ub, ib)


def kernel(user_idx, item_idx, user_embeddings, item_embeddings,
           user_biases, item_biases):
    uidx = user_idx.astype(jnp.int32)
    iidx = item_idx.astype(jnp.int32)
    uemb2 = jnp.reshape(user_embeddings, (NUM_USERS // 2, ROW_PAIR))
    iemb2 = jnp.reshape(item_embeddings, (NUM_ITEMS // 2, ROW_PAIR))
    ub = jnp.reshape(user_biases, (NUM_USERS,))
    ib = jnp.reshape(item_biases, (NUM_ITEMS,))
    return _run(uidx, iidx, uemb2, iemb2, ub, ib)


# speedup vs baseline: 1.0011x; 1.0011x over previous
import functools

import jax
import jax.numpy as jnp
from jax import lax
from jax.experimental import pallas as pl
from jax.experimental.pallas import tpu as pltpu
from jax.experimental.pallas import tpu_sc as plsc

NUM_USERS = 1000000
NUM_ITEMS = 1000000
EMB = 64
BATCH = 16384

NC = 2
NS = 16
NW = NC * NS
LANES = 16
B_PER_W = BATCH // NW
IDX_CHUNK = 128
N_IDX_CHUNKS = B_PER_W // IDX_CHUNK
ROW_PAIR = 2 * EMB
N_PASS = 2
B_PER_PASS = B_PER_W // N_PASS


def _sc_kernel_body(uidx_hbm, iidx_hbm, uemb2_hbm, iemb2_hbm, ub_hbm, ib_hbm,
                    out_hbm,
                    uidx_v, iidx_v, tu_v, ti_v, urows_v, irows_v,
                    ubv, ibv, out_v, sem):
    wid = lax.axis_index("s") * NC + lax.axis_index("c")
    base = pl.multiple_of(wid * B_PER_W, B_PER_W)

    pltpu.sync_copy(uidx_hbm.at[pl.ds(base, B_PER_W)], uidx_v)
    pltpu.sync_copy(iidx_hbm.at[pl.ds(base, B_PER_W)], iidx_v)

    bias_copies = []
    for k in range(N_IDX_CHUNKS):
        s = pl.ds(k * IDX_CHUNK, IDX_CHUNK)
        bias_copies.append(pltpu.async_copy(ub_hbm.at[uidx_v.at[s]],
                                            ubv.at[s], sem))
        bias_copies.append(pltpu.async_copy(ib_hbm.at[iidx_v.at[s]],
                                            ibv.at[s], sem))

    def halver(j, _):
        s = pl.ds(pl.multiple_of(j * LANES, LANES), LANES)
        tu_v[s] = lax.shift_right_logical(uidx_v[s], 1)
        ti_v[s] = lax.shift_right_logical(iidx_v[s], 1)
        return 0

    lax.fori_loop(0, B_PER_W // LANES, halver, 0)

    for c in bias_copies:
        c.wait()

    for p in range(N_PASS):
        poff = p * B_PER_PASS
        copies = []
        for k in range(B_PER_PASS // IDX_CHUNK):
            s_idx = pl.ds(poff + k * IDX_CHUNK, IDX_CHUNK)
            s_dst = pl.ds(k * IDX_CHUNK, IDX_CHUNK)
            copies.append(pltpu.async_copy(uemb2_hbm.at[tu_v.at[s_idx]],
                                           urows_v.at[s_dst], sem))
            copies.append(pltpu.async_copy(iemb2_hbm.at[ti_v.at[s_idx]],
                                           irows_v.at[s_dst], sem))
        for c in copies:
            c.wait()

        def chunk(j, _):
            start = pl.multiple_of(j * LANES, LANES)
            lanes = lax.iota(jnp.int32, LANES) + start
            gs = pl.ds(poff + start, LANES)
            uoff = (uidx_v[gs] & 1) * EMB
            ioff = (iidx_v[gs] & 1) * EMB
            acc = ubv[gs] + ibv[gs]
            for d in range(EMB):
                u = plsc.load_gather(urows_v, [lanes, uoff + d])
                v = plsc.load_gather(irows_v, [lanes, ioff + d])
                acc = acc + u * v
            out_v[gs] = acc
            return 0

        lax.fori_loop(0, B_PER_PASS // LANES, chunk, 0)

    pltpu.sync_copy(out_v, out_hbm.at[pl.ds(base, B_PER_W)])


@jax.jit
def _run(uidx, iidx, uemb, iemb, ub, ib):
    mesh = plsc.VectorSubcoreMesh(core_axis_name="c", subcore_axis_name="s")
    f = pl.kernel(
        _sc_kernel_body, mesh=mesh,
        out_type=jax.ShapeDtypeStruct((BATCH,), jnp.float32),
        scratch_types=[
            pltpu.VMEM((B_PER_W,), jnp.int32),
            pltpu.VMEM((B_PER_W,), jnp.int32),
            pltpu.VMEM((B_PER_W,), jnp.int32),
            pltpu.VMEM((B_PER_W,), jnp.int32),
            pltpu.VMEM((B_PER_PASS, ROW_PAIR), jnp.float32),
            pltpu.VMEM((B_PER_PASS, ROW_PAIR), jnp.float32),
            pltpu.VMEM((B_PER_W,), jnp.float32),
            pltpu.VMEM((B_PER_W,), jnp.float32),
            pltpu.VMEM((B_PER_W,), jnp.float32),
            pltpu.SemaphoreType.DMA,
        ],
        compiler_params=pltpu.CompilerParams(needs_layout_passes=False,
                                             use_tc_tiling_on_sc=True),
    )
    return f(uidx, iidx, uemb, iemb, ub, ib)


def kernel(user_idx, item_idx, user_embeddings, item_embeddings,
           user_biases, item_biases):
    uidx = user_idx.astype(jnp.int32)
    iidx = item_idx.astype(jnp.int32)
    uemb2 = jnp.reshape(user_embeddings, (NUM_USERS // 2, ROW_PAIR))
    iemb2 = jnp.reshape(item_embeddings, (NUM_ITEMS // 2, ROW_PAIR))
    ub = jnp.reshape(user_biases, (NUM_USERS,))
    ib = jnp.reshape(item_biases, (NUM_ITEMS,))
    return _run(uidx, iidx, uemb2, iemb2, ub, ib)
